# baseline (speedup 1.0000x reference)
import jax
import jax.numpy as jnp
from jax.experimental import pallas as pl
from jax.experimental.pallas import tpu as pltpu

_N_ACTIONS = 6


def _net_kernel(xs_ref, w1_ref, b1_ref, w2_ref, b2_ref, w3_ref, b3_ref,
                wf1_ref, bf1_ref, wf2_ref, bf2_ref, wex_ref, bex_ref,
                wh_ref, bh_ref, out_ref):
    f32 = jnp.float32
    bf16 = jnp.bfloat16
    tb = out_ref.shape[0]

    xs = xs_ref[...]
    w1 = w1_ref[...]
    b1 = b1_ref[...]

    h1 = {}
    for r in range(2):
        for u in range(2):
            pieces = []
            for dh in range(2):
                eh, p0 = (r + dh) % 2, (r + dh) // 2
                for dw in range(2):
                    ew, q0 = (u + dw) % 2, (u + dw) // 2
                    pieces.append(
                        xs[:, eh, ew, p0:p0 + 10, q0:q0 + 10, :])
            pat = jnp.concatenate(pieces, axis=-1)
            hru = jnp.maximum(
                jnp.dot(pat.reshape(tb * 100, 256), w1,
                        preferred_element_type=f32) + b1, 0.0)
            h1[(r, u)] = hru.astype(bf16).reshape(tb, 10, 10, 32)

    pieces2 = []
    for kh in range(4):
        a, r = kh // 2, kh % 2
        for kw in range(4):
            b, u = kw // 2, kw % 2
            pieces2.append(h1[(r, u)][:, a:a + 9, b:b + 9, :])
    pat2 = jnp.concatenate(pieces2, axis=-1)
    h2 = jnp.maximum(
        jnp.dot(pat2.reshape(tb * 81, 512), w2_ref[...],
                preferred_element_type=f32) + b2_ref[...], 0.0)
    h2 = h2.astype(bf16)

    h2r = h2.reshape(tb, 9, 9, 64)
    pat3 = jnp.concatenate(
        [h2r[:, kh:kh + 7, kw:kw + 7, :]
         for kh in range(3) for kw in range(3)], axis=-1)
    h3 = jnp.maximum(
        jnp.dot(pat3.reshape(tb * 49, 576), w3_ref[...],
                preferred_element_type=f32) + b3_ref[...], 0.0)
    h3 = h3.astype(bf16)

    h3r = h3.reshape(tb, 49, 64)
    hf = jnp.concatenate([h3r[:, q, :] for q in range(49)], axis=-1)
    h4 = jnp.maximum(
        jnp.dot(hf, wf1_ref[...], preferred_element_type=f32)
        + bf1_ref[...], 0.0).astype(bf16)
    h5 = jnp.maximum(
        jnp.dot(h4, wf2_ref[...], preferred_element_type=f32)
        + bf2_ref[...], 0.0)

    rr = jnp.maximum(
        jnp.dot(h5.astype(bf16), wex_ref[...], preferred_element_type=f32)
        + bex_ref[...], 0.0)
    x_v = h5 + rr[:, :448]
    x_pi = h5 + rr[:, 448:]

    lhs = jnp.concatenate([x_v, x_pi], axis=0).astype(bf16)
    head = (jnp.dot(lhs, wh_ref[...], preferred_element_type=f32)
            + bh_ref[...])
    vals = head[:tb, :]
    logits = head[tb:, :]

    col = jax.lax.broadcasted_iota(jnp.int32, logits.shape, 1)
    lmask = jnp.where(col < _N_ACTIONS, logits, jnp.float32(-1e30))
    m = jnp.max(lmask, axis=-1, keepdims=True)
    e = jnp.exp(lmask - m)
    probs = e * pl.reciprocal(jnp.sum(e, axis=-1, keepdims=True), approx=False)

    out_ref[...] = jnp.where(col < _N_ACTIONS, probs,
                             jnp.where(col < _N_ACTIONS + 2, vals, 0.0))


def kernel(x, w_c1, b_c1, w_c2, b_c2, w_c3, b_c3, w_fc1, b_fc1,
           w_fc2, b_fc2, w_extra, b_extra, w_heads, b_heads):
    B = x.shape[0]
    bf16 = jnp.bfloat16
    head_w = w_heads.shape[1]

    xb = jnp.pad(x.astype(bf16), ((0, 0), (0, 0), (0, 4), (0, 4)))
    xsp = (xb.reshape(B, 4, 11, 2, 4, 11, 2, 4)
             .transpose(0, 3, 6, 2, 5, 4, 7, 1)
             .reshape(B, 2, 2, 11, 11, 64))

    w1r = (w_c1.reshape(2, 4, 2, 4, 4, 32)
               .transpose(0, 2, 1, 3, 4, 5)
               .reshape(256, 32))

    tb = next(t for t in (32, 16, 8, 4, 2, 1) if B % t == 0)

    weights = [w1r.astype(bf16), b_c1,
               w_c2.astype(bf16), b_c2,
               w_c3.astype(bf16), b_c3,
               w_fc1.astype(bf16), b_fc1,
               w_fc2.astype(bf16), b_fc2,
               w_extra.astype(bf16), b_extra,
               w_heads.astype(bf16), b_heads]

    in_specs = [pl.BlockSpec((tb, 2, 2, 11, 11, 64),
                             lambda i: (i, 0, 0, 0, 0, 0))]
    in_specs += [pl.BlockSpec(w.shape, lambda i: (0,) * w.ndim)
                 for w in weights]

    out = pl.pallas_call(
        _net_kernel,
        out_shape=jax.ShapeDtypeStruct((B, head_w), jnp.float32),
        grid=(B // tb,),
        in_specs=in_specs,
        out_specs=pl.BlockSpec((tb, head_w), lambda i: (i, 0)),
        compiler_params=pltpu.CompilerParams(
            dimension_semantics=("parallel",)),
    )(xsp, *weights)

    probs = out[:, :_N_ACTIONS]
    int_value = out[:, _N_ACTIONS:_N_ACTIONS + 1]
    ext_value = out[:, _N_ACTIONS + 1:_N_ACTIONS + 2]
    return probs, int_value, ext_value
